# Initial kernel scaffold; baseline (speedup 1.0000x reference)
import jax
import jax.numpy as jnp
from jax.experimental import pallas as pl


def kernel(x, edge_index, W, b):
    raise NotImplementedError("write your pallas kernel here")


# speedup vs baseline: 1903.0813x; 1903.0813x over previous
import functools

import jax
import jax.numpy as jnp
from jax import lax
from jax.experimental import pallas as pl
from jax.experimental.pallas import tpu as pltpu
from jax.experimental.pallas import tpu_sc as plsc

B, C, N, K = 1, 256, 10000, 16
NP = 10240
NUM_TILES = 32
CT = C // NUM_TILES
NB = 512
NGROUPS = NB // 16
NBLOCKS = NP // NB


def _sc_gather_max(x_flat, idx_flat):
    mesh = plsc.VectorSubcoreMesh(core_axis_name="c", subcore_axis_name="s")

    @functools.partial(
        pl.kernel,
        out_type=jax.ShapeDtypeStruct((C * NP,), jnp.float32),
        mesh=mesh,
        compiler_params=pltpu.CompilerParams(needs_layout_passes=False),
        scratch_types=[
            pltpu.VMEM((CT * NP,), jnp.float32),
            pltpu.VMEM((K * NB,), jnp.int32),
            pltpu.VMEM((CT * NB,), jnp.float32),
        ],
    )
    def sc_kernel(x_hbm, idx_hbm, h_hbm, xv, idxv, hv):
        num_cores = 2
        wid = lax.axis_index("s") * num_cores + lax.axis_index("c")
        c0 = wid * CT
        pltpu.sync_copy(x_hbm.at[pl.ds(c0 * NP, CT * NP)], xv)

        def node_block(blk, carry):
            n0 = blk * NB
            for kk in range(K):
                pltpu.sync_copy(idx_hbm.at[pl.ds(kk * NP + n0, NB)],
                                idxv.at[pl.ds(kk * NB, NB)])

            def group(g, carry2):
                off = g * 16
                ivs = [idxv[pl.ds(kk * NB + off, 16)] for kk in range(K)]
                for c in range(CT):
                    base = jnp.full((16,), c * NP, jnp.int32)
                    m = plsc.load_gather(xv, [base + ivs[0]])
                    for kk in range(1, K):
                        m = jnp.maximum(m, plsc.load_gather(xv, [base + ivs[kk]]))
                    hv[pl.ds(c * NB + off, 16)] = xv[pl.ds(c * NP + n0 + off, 16)] + m
                return carry2

            lax.fori_loop(0, NGROUPS, group, 0)
            for c in range(CT):
                pltpu.sync_copy(hv.at[pl.ds(c * NB, NB)],
                                h_hbm.at[pl.ds((c0 + c) * NP + n0, NB)])
            return carry

        lax.fori_loop(0, NBLOCKS, node_block, 0)

    return sc_kernel(x_flat, idx_flat)


def _tc_mlp(h, W, b2d):
    BN = 1024

    def body(w_ref, h_ref, b_ref, o_ref):
        acc = jnp.dot(w_ref[...], h_ref[...], preferred_element_type=jnp.float32)
        o_ref[...] = jnp.maximum(acc + b_ref[...], 0.0)

    return pl.pallas_call(
        body,
        grid=(pl.cdiv(N, BN),),
        in_specs=[
            pl.BlockSpec((C, C), lambda i: (0, 0)),
            pl.BlockSpec((C, BN), lambda i: (0, i)),
            pl.BlockSpec((C, 1), lambda i: (0, 0)),
        ],
        out_specs=pl.BlockSpec((C, BN), lambda i: (0, i)),
        out_shape=jax.ShapeDtypeStruct((C, N), jnp.float32),
    )(W, h, b2d)


def kernel(x, edge_index, W, b):
    x2d = jnp.pad(x[0, :, :, 0], ((0, 0), (0, NP - N)))
    idx_t = jnp.pad(edge_index[0, 0].astype(jnp.int32).T,
                    ((0, 0), (0, NP - N)))
    h = _sc_gather_max(x2d.reshape(-1), idx_t.reshape(-1)).reshape(C, NP)
    out = _tc_mlp(h, W, b[:, None])
    return out[None, :, :, None]


# speedup vs baseline: 3388.8946x; 1.7807x over previous
import functools

import jax
import jax.numpy as jnp
from jax import lax
from jax.experimental import pallas as pl
from jax.experimental.pallas import tpu as pltpu
from jax.experimental.pallas import tpu_sc as plsc

B, C, N, K = 1, 256, 10000, 16
NP = 10240
NUM_TILES = 32
CT = C // NUM_TILES
NB = 512
NGROUPS = NB // 16
NBLOCKS = NP // NB


def _sc_gather_max(x_flat, idx_flat):
    mesh = plsc.VectorSubcoreMesh(core_axis_name="c", subcore_axis_name="s")

    @functools.partial(
        pl.kernel,
        out_type=jax.ShapeDtypeStruct((C * NP,), jnp.float32),
        mesh=mesh,
        compiler_params=pltpu.CompilerParams(needs_layout_passes=False),
        scratch_types=[
            pltpu.VMEM((CT * NP,), jnp.float32),
            pltpu.VMEM((K * NB,), jnp.int32),
            pltpu.VMEM((CT * NB,), jnp.float32),
        ],
    )
    def sc_kernel(x_hbm, idx_hbm, h_hbm, xv, idxv, hv):
        num_cores = 2
        wid = lax.axis_index("s") * num_cores + lax.axis_index("c")
        c0 = wid * CT
        pltpu.sync_copy(x_hbm.at[pl.ds(c0 * NP, CT * NP)], xv)

        def node_block(blk, carry):
            n0 = blk * NB
            pltpu.sync_copy(idx_hbm.at[pl.ds(blk * K * NB, K * NB)], idxv)

            @plsc.parallel_loop(0, NGROUPS)
            def group(g):
                off = g * 16
                ivs = [idxv[pl.ds(kk * NB + off, 16)] for kk in range(K)]
                for c in range(CT):
                    base = jnp.full((16,), c * NP, jnp.int32)
                    gs = [plsc.load_gather(xv, [base + ivs[kk]]) for kk in range(K)]
                    while len(gs) > 1:
                        gs = [jnp.maximum(gs[2 * i], gs[2 * i + 1])
                              for i in range(len(gs) // 2)]
                    hv[pl.ds(c * NB + off, 16)] = xv[pl.ds(c * NP + n0 + off, 16)] + gs[0]
            pltpu.sync_copy(hv, h_hbm.at[pl.ds(blk * C * NB + c0 * NB, CT * NB)])
            return carry

        lax.fori_loop(0, NBLOCKS, node_block, 0)

    return sc_kernel(x_flat, idx_flat)


def _tc_mlp(h_blocked, W, b2d):
    def body(w_ref, h_ref, b_ref, o_ref):
        acc = jnp.dot(w_ref[...], h_ref[0], preferred_element_type=jnp.float32)
        o_ref[...] = jnp.maximum(acc + b_ref[...], 0.0)

    return pl.pallas_call(
        body,
        grid=(NBLOCKS,),
        in_specs=[
            pl.BlockSpec((C, C), lambda i: (0, 0)),
            pl.BlockSpec((1, C, NB), lambda i: (i, 0, 0)),
            pl.BlockSpec((C, 1), lambda i: (0, 0)),
        ],
        out_specs=pl.BlockSpec((C, NB), lambda i: (0, i)),
        out_shape=jax.ShapeDtypeStruct((C, N), jnp.float32),
    )(W, h_blocked, b2d)


def kernel(x, edge_index, W, b):
    x2d = jnp.pad(x[0, :, :, 0], ((0, 0), (0, NP - N)))
    idx_t = jnp.pad(edge_index[0, 0].astype(jnp.int32).T,
                    ((0, 0), (0, NP - N)))
    idx_b = idx_t.reshape(K, NBLOCKS, NB).transpose(1, 0, 2).reshape(-1)
    h = _sc_gather_max(x2d.reshape(-1), idx_b)
    out = _tc_mlp(h.reshape(NBLOCKS, C, NB), W, b[:, None])
    return out[None, :, :, None]
